# baseline (speedup 1.0000x reference)
import jax
import jax.numpy as jnp
from jax.experimental import pallas as pl

N_ATOM = 10000; E_ATOM = 160000; N_FG = 2500; E_FG = 20000; B = 256
HID = 16; STEP = 2


def _bn(x, g, b):
    mu = jnp.mean(x, axis=0, keepdims=True)
    var = jnp.var(x, axis=0, keepdims=True)
    return g * (x - mu) / jnp.sqrt(var + 1e-5) + b


def _encoder(x, p, pre):
    h = jax.nn.relu(_bn(x @ p[pre + '_W1'] + p[pre + '_b1'], p[pre + '_g1'], p[pre + '_be1']))
    h = jax.nn.relu(_bn(h @ p[pre + '_W2'] + p[pre + '_b2'], p[pre + '_g2'], p[pre + '_be2']))
    return h


def _mpnn(h, ef, src, dst, We, be, n_nodes):
    Wmat = (ef @ We + be).reshape(-1, HID, HID)
    msg = jnp.einsum('eij,ej->ei', Wmat, h[src])
    agg = jax.ops.segment_sum(msg, dst, num_segments=n_nodes)
    return agg + h


def _gru(x, h, Wx, Wh, bx, bh):
    gx = x @ Wx + bx
    gh = h @ Wh + bh
    xr, xz, xn = jnp.split(gx, 3, axis=1)
    hr, hz, hn = jnp.split(gh, 3, axis=1)
    r = jax.nn.sigmoid(xr + hr)
    z = jax.nn.sigmoid(xz + hz)
    n = jnp.tanh(xn + r * hn)
    return (1.0 - z) * n + z * h


def _readout(h, gid, nb):
    s = jax.ops.segment_sum(h, gid, num_segments=nb)
    m = jax.ops.segment_max(h, gid, num_segments=nb)
    m = jnp.where(jnp.isfinite(m), m, 0.0)
    return jnp.concatenate([s, m], axis=1)


def _identity_pallas(x):
    def body(x_ref, o_ref):
        o_ref[...] = x_ref[...]
    return pl.pallas_call(body, out_shape=jax.ShapeDtypeStruct(x.shape, x.dtype))(x)


def kernel(af, bf, fnf, fef, mf, labels, edge_index_atom, edge_index_fg, atom2fg, atom_gid, fg_gid, params):
    p = params
    uaf = _encoder(af, p, 'ae')
    ufnf = _encoder(fnf, p, 'fe')
    a_src, a_dst = edge_index_atom[0], edge_index_atom[1]
    f_src, f_dst = edge_index_fg[0], edge_index_fg[1]
    for _ in range(STEP):
        ufnm = _mpnn(ufnf, fef, f_src, f_dst, p['mp_fg_We'], p['mp_fg_be'], N_FG)
        uam = _mpnn(uaf, bf, a_src, a_dst, p['mp_at_We'], p['mp_at_be'], N_ATOM)
        agg_uam = jax.ops.segment_sum(uam, atom2fg, num_segments=N_FG)
        ufnm = jnp.concatenate([ufnm, agg_uam], axis=1)
        ufnf = _gru(ufnm, ufnf, p['gru_fg_Wx'], p['gru_fg_Wh'], p['gru_fg_bx'], p['gru_fg_bh'])
        uaf = _gru(uam, uaf, p['gru_at_Wx'], p['gru_at_Wh'], p['gru_at_bx'], p['gru_at_bh'])
    fg_rd = _readout(ufnf, fg_gid, B)
    at_rd = _readout(uaf, atom_gid, B)
    atom_rep = jnp.concatenate([at_rd, mf], axis=1)
    ss_rep = jnp.concatenate([fg_rd, mf], axis=1)
    a_sq = atom_rep @ p['cr_Wa'] + p['cr_ba']
    f_sq = ss_rep @ p['cr_Wf'] + p['cr_bf']
    joint = jnp.concatenate([a_sq, f_sq], axis=1)
    w = jax.nn.relu(joint @ p['cr_Wm1'] + p['cr_bm1']) @ p['cr_Wm2'] + p['cr_bm2']
    w = jax.nn.sigmoid(w)
    atom_rep = atom_rep * w[:, 0:1]
    ss_rep = ss_rep * w[:, 1:2]
    pred_a = jax.nn.relu(atom_rep @ p['out_a_W1'] + p['out_a_b1']) @ p['out_a_W2'] + p['out_a_b2']
    pred_f = jax.nn.relu(ss_rep @ p['out_f_W1'] + p['out_f_b1']) @ p['out_f_W2'] + p['out_f_b2']
    pred = jnp.concatenate([pred_a, pred_f], axis=1)
    return _identity_pallas(pred)


# speedup vs baseline: 1.1569x; 1.1569x over previous
import functools

import jax
import jax.numpy as jnp
from jax import lax
from jax.experimental import pallas as pl
from jax.experimental.pallas import tpu as pltpu
from jax.experimental.pallas import tpu_sc as plsc

N_ATOM = 10000; E_ATOM = 160000; N_FG = 2500; E_FG = 20000; B = 256
HID = 16; STEP = 2

NC, NS, LANES = 2, 16, 16
NW = NC * NS

NAP = 10240
NFP = 2560
EAP = 163840
EFP = 20480
ACH = 40
FCH = 5
NCH = 5

F32 = jnp.float32
NEG = -1e30


@functools.lru_cache(maxsize=None)
def _sc_mesh():
    return plsc.VectorSubcoreMesh(core_axis_name="c", subcore_axis_name="s",
                                  num_cores=NC, num_subcores=NS)


def _enc_one(x, W1, b1, g1, be1, W2, b2, g2, be2):
    h = jnp.dot(x, W1, preferred_element_type=F32) + b1
    mu = jnp.mean(h, axis=0, keepdims=True)
    var = jnp.mean((h - mu) ** 2, axis=0, keepdims=True)
    h = jax.nn.relu(g1 * (h - mu) / jnp.sqrt(var + 1e-5) + be1)
    h2 = jnp.dot(h, W2, preferred_element_type=F32) + b2
    mu2 = jnp.mean(h2, axis=0, keepdims=True)
    var2 = jnp.mean((h2 - mu2) ** 2, axis=0, keepdims=True)
    return jax.nn.relu(g2 * (h2 - mu2) / jnp.sqrt(var2 + 1e-5) + be2)


def _encoder_body(af_ref, fnf_ref, *refs):
    (aW1, ab1, ag1, abe1, aW2, ab2, ag2, abe2,
     fW1, fb1, fg1, fbe1, fW2, fb2, fg2, fbe2, uaf_ref, ufnf_ref) = refs
    ua = _enc_one(af_ref[...], aW1[...], ab1[...], ag1[...], abe1[...],
                  aW2[...], ab2[...], ag2[...], abe2[...])
    uaf_ref[0:N_ATOM, :] = ua
    uaf_ref[N_ATOM:NAP, :] = jnp.zeros((NAP - N_ATOM, HID), F32)
    uf = _enc_one(fnf_ref[...], fW1[...], fb1[...], fg1[...], fbe1[...],
                  fW2[...], fb2[...], fg2[...], fbe2[...])
    ufnf_ref[0:N_FG, :] = uf
    ufnf_ref[N_FG:NFP, :] = jnp.zeros((NFP - N_FG, HID), F32)


def _run_encoders(af, fnf, p):
    outs = pl.pallas_call(
        _encoder_body,
        out_shape=(jax.ShapeDtypeStruct((NAP, HID), F32),
                   jax.ShapeDtypeStruct((NFP, HID), F32)),
    )(af, fnf,
      p['ae_W1'], p['ae_b1'].reshape(1, -1), p['ae_g1'].reshape(1, -1), p['ae_be1'].reshape(1, -1),
      p['ae_W2'], p['ae_b2'].reshape(1, -1), p['ae_g2'].reshape(1, -1), p['ae_be2'].reshape(1, -1),
      p['fe_W1'], p['fe_b1'].reshape(1, -1), p['fe_g1'].reshape(1, -1), p['fe_be1'].reshape(1, -1),
      p['fe_W2'], p['fe_b2'].reshape(1, -1), p['fe_g2'].reshape(1, -1), p['fe_be2'].reshape(1, -1))
    return outs


def _gather_body(uaf_hbm, asrc_hbm, ufnf_hbm, fsrc_hbm, hsat_hbm, hsfg_hbm,
                 idxa_v, rowsa_v, idxf_v, rowsf_v, sem):
    c = lax.axis_index("c")
    s = lax.axis_index("s")
    wid = s * NC + c
    pltpu.sync_copy(asrc_hbm.at[wid], idxa_v)
    pltpu.sync_copy(fsrc_hbm.at[wid], idxf_v)

    def agroup(g, carry):
        descs = []
        for b in range(8):
            j = g * 8 + b
            d = pltpu.async_copy(uaf_hbm.at[idxa_v.at[j]],
                                 rowsa_v.at[pl.ds(j * 128, 128)], sem)
            descs.append(d)
        for d in descs:
            d.wait()
        return carry

    lax.fori_loop(0, ACH // 8, agroup, 0)

    descs = []
    for j in range(FCH):
        d = pltpu.async_copy(ufnf_hbm.at[idxf_v.at[j]],
                             rowsf_v.at[pl.ds(j * 128, 128)], sem)
        descs.append(d)
    for d in descs:
        d.wait()

    pltpu.sync_copy(rowsa_v, hsat_hbm.at[pl.ds(wid * (ACH * 128), ACH * 128)])
    pltpu.sync_copy(rowsf_v, hsfg_hbm.at[pl.ds(wid * (FCH * 128), FCH * 128)])


@functools.lru_cache(maxsize=None)
def _gather_call():
  return pl.kernel(
    _gather_body,
    out_type=(jax.ShapeDtypeStruct((EAP, HID), F32),
              jax.ShapeDtypeStruct((EFP, HID), F32)),
    mesh=_sc_mesh(),
    scratch_types=[
        pltpu.VMEM((ACH, 128), jnp.int32),
        pltpu.VMEM((ACH * 128, HID), F32),
        pltpu.VMEM((FCH, 128), jnp.int32),
        pltpu.VMEM((FCH * 128, HID), F32),
        pltpu.SemaphoreType.DMA,
    ],
    compiler_params=pltpu.CompilerParams(use_tc_tiling_on_sc=False),
)


def _msg_body(hs_ref, bfx_ref, wcat_ref, o_ref):
    hs = hs_ref[...]
    bfx = bfx_ref[...]
    C = jnp.dot(hs, wcat_ref[...], preferred_element_type=F32)
    acc = bfx[:, 0:1] * C[:, 0:HID]
    for k in range(1, 17):
        acc = acc + bfx[:, k:k + 1] * C[:, k * HID:(k + 1) * HID]
    o_ref[...] = acc


def _run_msg(hs, bfx, wcat, epad, eb):
    grid = epad // eb
    return pl.pallas_call(
        _msg_body,
        grid=(grid,),
        in_specs=[pl.BlockSpec((eb, HID), lambda i: (i, 0)),
                  pl.BlockSpec((eb, 17), lambda i: (i, 0)),
                  pl.BlockSpec((HID, 17 * HID), lambda i: (0, 0))],
        out_specs=pl.BlockSpec((eb, HID), lambda i: (i, 0)),
        out_shape=jax.ShapeDtypeStruct((epad, HID), F32),
    )(hs, bfx, wcat)


def _scatter_body(msgat_hbm, adst_hbm, msgfg_hbm, fdst_hbm, zat_hbm, zfg_hbm,
                  aggat_hbm, aggfg_hbm,
                  msg_v, idxa_v, msgf_v, idxf_v, acc_at, acc_fg):
    c = lax.axis_index("c")
    s = lax.axis_index("s")
    wid = s * NC + c
    pltpu.sync_copy(zat_hbm, acc_at.at[pl.ds(s * 640, 640)])
    pltpu.sync_copy(zfg_hbm, acc_fg.at[pl.ds(s * 160, 160)])
    pltpu.sync_copy(msgat_hbm.at[pl.ds(wid * (ACH * 128), ACH * 128)], msg_v)
    pltpu.sync_copy(adst_hbm.at[wid], idxa_v)
    pltpu.sync_copy(msgfg_hbm.at[pl.ds(wid * (FCH * 128), FCH * 128)], msgf_v)
    pltpu.sync_copy(fdst_hbm.at[wid], idxf_v)
    plsc.subcore_barrier()

    def abody(j, carry):
        pltpu.sync_copy(msg_v.at[pl.ds(j * 128, 128)],
                        acc_at.at[idxa_v.at[j]], add=True)
        return carry

    lax.fori_loop(0, ACH, abody, 0)
    for j in range(FCH):
        pltpu.sync_copy(msgf_v.at[pl.ds(j * 128, 128)],
                        acc_fg.at[idxf_v.at[j]], add=True)
    plsc.subcore_barrier()
    pltpu.sync_copy(acc_at.at[pl.ds(s * 640, 640)],
                    aggat_hbm.at[c, pl.ds(s * 640, 640)])
    pltpu.sync_copy(acc_fg.at[pl.ds(s * 160, 160)],
                    aggfg_hbm.at[c, pl.ds(s * 160, 160)])


@functools.lru_cache(maxsize=None)
def _scatter_call():
  return pl.kernel(
    _scatter_body,
    out_type=(jax.ShapeDtypeStruct((NC, NAP, HID), F32),
              jax.ShapeDtypeStruct((NC, NFP, HID), F32)),
    mesh=_sc_mesh(),
    scratch_types=[
        pltpu.VMEM((ACH * 128, HID), F32),
        pltpu.VMEM((ACH, 128), jnp.int32),
        pltpu.VMEM((FCH * 128, HID), F32),
        pltpu.VMEM((FCH, 128), jnp.int32),
        pltpu.VMEM_SHARED((NAP, HID), F32),
        pltpu.VMEM_SHARED((NFP, HID), F32),
    ],
    compiler_params=pltpu.CompilerParams(use_tc_tiling_on_sc=False),
)


def _a2f_body(uam_hbm, a2f_hbm, zfg_hbm, aggm_hbm, uam_v, idx_v, acc_fg):
    c = lax.axis_index("c")
    s = lax.axis_index("s")
    wid = s * NC + c
    pltpu.sync_copy(zfg_hbm, acc_fg.at[pl.ds(s * 160, 160)])
    pltpu.sync_copy(uam_hbm.at[pl.ds(wid * (NCH * 64), NCH * 64)], uam_v)
    pltpu.sync_copy(a2f_hbm.at[wid], idx_v)
    plsc.subcore_barrier()
    for j in range(NCH):
        pltpu.sync_copy(uam_v.at[pl.ds(j * 64, 64)],
                        acc_fg.at[idx_v.at[j]], add=True)
    plsc.subcore_barrier()
    pltpu.sync_copy(acc_fg.at[pl.ds(s * 160, 160)],
                    aggm_hbm.at[c, pl.ds(s * 160, 160)])


@functools.lru_cache(maxsize=None)
def _a2f_call():
  return pl.kernel(
    _a2f_body,
    out_type=jax.ShapeDtypeStruct((NC, NFP, HID), F32),
    mesh=_sc_mesh(),
    scratch_types=[
        pltpu.VMEM((NCH * 64, HID), F32),
        pltpu.VMEM((NCH, 64), jnp.int32),
        pltpu.VMEM_SHARED((NFP, HID), F32),
    ],
    compiler_params=pltpu.CompilerParams(use_tc_tiling_on_sc=False),
)


def _gru_core(x, h, Wx, Wh, bx, bh):
    gx = jnp.dot(x, Wx, preferred_element_type=F32) + bx
    gh = jnp.dot(h, Wh, preferred_element_type=F32) + bh
    r = jax.nn.sigmoid(gx[:, 0:HID] + gh[:, 0:HID])
    z = jax.nn.sigmoid(gx[:, HID:2 * HID] + gh[:, HID:2 * HID])
    n = jnp.tanh(gx[:, 2 * HID:3 * HID] + r * gh[:, 2 * HID:3 * HID])
    return (1.0 - z) * n + z * h


def _gru_at_body(agg_ref, uaf_ref, Wx_ref, Wh_ref, bx_ref, bh_ref,
                 uam_ref, uafn_ref):
    uaf = uaf_ref[...]
    uam = agg_ref[0] + agg_ref[1] + uaf
    uam_ref[...] = uam
    uafn_ref[...] = _gru_core(uam, uaf, Wx_ref[...], Wh_ref[...],
                              bx_ref[...], bh_ref[...])


def _run_gru_at(agg, uaf, p):
    return pl.pallas_call(
        _gru_at_body,
        out_shape=(jax.ShapeDtypeStruct((NAP, HID), F32),
                   jax.ShapeDtypeStruct((NAP, HID), F32)),
    )(agg, uaf, p['gru_at_Wx'], p['gru_at_Wh'],
      p['gru_at_bx'].reshape(1, -1), p['gru_at_bh'].reshape(1, -1))


def _gru_fg_body(aggfg_ref, aggm_ref, ufnf_ref, Wx_ref, Wh_ref, bx_ref, bh_ref,
                 ufnfn_ref):
    ufnf = ufnf_ref[...]
    m1 = aggfg_ref[0] + aggfg_ref[1] + ufnf
    m2 = aggm_ref[0] + aggm_ref[1]
    ufnm = jnp.concatenate([m1, m2], axis=1)
    ufnfn_ref[...] = _gru_core(ufnm, ufnf, Wx_ref[...], Wh_ref[...],
                               bx_ref[...], bh_ref[...])


def _run_gru_fg(aggfg, aggm, ufnf, p):
    return pl.pallas_call(
        _gru_fg_body,
        out_shape=jax.ShapeDtypeStruct((NFP, HID), F32),
    )(aggfg, aggm, ufnf, p['gru_fg_Wx'], p['gru_fg_Wh'],
      p['gru_fg_bx'].reshape(1, -1), p['gru_fg_bh'].reshape(1, -1))


def _readout_body(gid_ref, h_ref, s_ref, m_ref):
    i = pl.program_id(0)

    @pl.when(i == 0)
    def _():
        s_ref[...] = jnp.zeros((HID, B), F32)
        m_ref[...] = jnp.full((HID, B), NEG, F32)

    gcol = gid_ref[0]
    segs = lax.broadcasted_iota(jnp.int32, (1, B), 1)
    onehot = (gcol == segs).astype(F32)
    h = h_ref[...]
    s_ref[...] += lax.dot_general(h, onehot, (((0,), (0,)), ((), ())),
                                  preferred_element_type=F32)
    mask3 = (gcol[:, :, None] == lax.broadcasted_iota(jnp.int32, (1, 1, B), 2))
    h3 = h[:, :, None]
    contrib = jnp.max(jnp.where(mask3, h3, NEG), axis=0)
    m_ref[...] = jnp.maximum(m_ref[...], contrib)


def _run_readout(h, gid_col, npad):
    grid = npad // 128
    return pl.pallas_call(
        _readout_body,
        grid=(grid,),
        in_specs=[pl.BlockSpec((1, 128, 1), lambda i: (i, 0, 0)),
                  pl.BlockSpec((128, HID), lambda i: (i, 0))],
        out_specs=(pl.BlockSpec((HID, B), lambda i: (0, 0)),
                   pl.BlockSpec((HID, B), lambda i: (0, 0))),
        out_shape=(jax.ShapeDtypeStruct((HID, B), F32),
                   jax.ShapeDtypeStruct((HID, B), F32)),
    )(gid_col, h)


def _head_body(ats_ref, atm_ref, fgs_ref, fgm_ref, mf_ref, *refs):
    (Wa, ba, Wf, bf_, Wm1, bm1, Wm2, bm2,
     oaW1, oab1, oaW2, oab2, ofW1, ofb1, ofW2, ofb2, out_ref) = refs
    eye = (lax.broadcasted_iota(jnp.int32, (B, B), 0)
           == lax.broadcasted_iota(jnp.int32, (B, B), 1)).astype(F32)

    def tr(x_ref):
        return lax.dot_general(eye, x_ref[...], (((1,), (1,)), ((), ())),
                               preferred_element_type=F32)

    at_s = tr(ats_ref)
    at_m = tr(atm_ref)
    fg_s = tr(fgs_ref)
    fg_m = tr(fgm_ref)
    at_m = jnp.where(at_m > 0.5 * NEG, at_m, 0.0)
    fg_m = jnp.where(fg_m > 0.5 * NEG, fg_m, 0.0)
    mf = mf_ref[...]
    atom_rep = jnp.concatenate([at_s, at_m, mf], axis=1)
    ss_rep = jnp.concatenate([fg_s, fg_m, mf], axis=1)
    a_sq = jnp.dot(atom_rep, Wa[...], preferred_element_type=F32) + ba[...]
    f_sq = jnp.dot(ss_rep, Wf[...], preferred_element_type=F32) + bf_[...]
    joint = jnp.concatenate([a_sq, f_sq], axis=1)
    w = jnp.dot(jax.nn.relu(jnp.dot(joint, Wm1[...],
                                    preferred_element_type=F32) + bm1[...]),
                Wm2[...], preferred_element_type=F32) + bm2[...]
    w = jax.nn.sigmoid(w)
    atom_rep = atom_rep * w[:, 0:1]
    ss_rep = ss_rep * w[:, 1:2]
    pa = jnp.dot(jax.nn.relu(jnp.dot(atom_rep, oaW1[...],
                                     preferred_element_type=F32) + oab1[...]),
                 oaW2[...], preferred_element_type=F32) + oab2[...]
    pf = jnp.dot(jax.nn.relu(jnp.dot(ss_rep, ofW1[...],
                                     preferred_element_type=F32) + ofb1[...]),
                 ofW2[...], preferred_element_type=F32) + ofb2[...]
    out_ref[...] = jnp.concatenate([pa, pf], axis=1)


def _run_head(at_s, at_m, fg_s, fg_m, mf, p):
    return pl.pallas_call(
        _head_body,
        out_shape=jax.ShapeDtypeStruct((B, 2), F32),
    )(at_s, at_m, fg_s, fg_m, mf,
      p['cr_Wa'], p['cr_ba'].reshape(1, -1), p['cr_Wf'], p['cr_bf'].reshape(1, -1),
      p['cr_Wm1'], p['cr_bm1'].reshape(1, -1), p['cr_Wm2'], p['cr_bm2'].reshape(1, -1),
      p['out_a_W1'], p['out_a_b1'].reshape(1, -1), p['out_a_W2'], p['out_a_b2'].reshape(1, -1),
      p['out_f_W1'], p['out_f_b1'].reshape(1, -1), p['out_f_W2'], p['out_f_b2'].reshape(1, -1))


def _pad1(x, n, val):
    return jnp.pad(x, (0, n - x.shape[0]), constant_values=val)


def _wcat(We, be):
    wex = jnp.concatenate([We, be.reshape(1, -1)], axis=0)
    a = wex.reshape(17, HID, HID)
    return a.transpose(2, 0, 1).reshape(HID, 17 * HID)


def kernel(af, bf, fnf, fef, mf, labels, edge_index_atom, edge_index_fg,
           atom2fg, atom_gid, fg_gid, params):
    p = params
    a_src = _pad1(edge_index_atom[0], EAP, 0).reshape(NW, ACH, 128)
    a_dst = _pad1(edge_index_atom[1], EAP, 0).reshape(NW, ACH, 128)
    f_src = _pad1(edge_index_fg[0], EFP, 0).reshape(NW, FCH, 128)
    f_dst = _pad1(edge_index_fg[1], EFP, 0).reshape(NW, FCH, 128)
    a2f = _pad1(atom2fg, NAP, 0).reshape(NW, NCH, 64)
    bfx = jnp.pad(jnp.concatenate([bf, jnp.ones((E_ATOM, 1), F32)], axis=1),
                  ((0, EAP - E_ATOM), (0, 0)))
    fefx = jnp.pad(jnp.concatenate([fef, jnp.ones((E_FG, 1), F32)], axis=1),
                   ((0, EFP - E_FG), (0, 0)))
    wcat_at = _wcat(p['mp_at_We'], p['mp_at_be'])
    wcat_fg = _wcat(p['mp_fg_We'], p['mp_fg_be'])
    agid_col = _pad1(atom_gid, NAP, B + 8).reshape(NAP // 128, 128, 1)
    fgid_col = _pad1(fg_gid, NFP, B + 8).reshape(NFP // 128, 128, 1)
    z640 = jnp.zeros((640, HID), F32)
    z160 = jnp.zeros((160, HID), F32)

    uaf, ufnf = _run_encoders(af, fnf, p)

    for _ in range(STEP):
        hs_at, hs_fg = _gather_call()(uaf, a_src, ufnf, f_src)
        msg_at = _run_msg(hs_at, bfx, wcat_at, EAP, 4096)
        msg_fg = _run_msg(hs_fg, fefx, wcat_fg, EFP, 4096)
        agg_at, agg_fg = _scatter_call()(msg_at, a_dst, msg_fg, f_dst, z640, z160)
        uam, uaf = _run_gru_at(agg_at, uaf, p)
        aggm = _a2f_call()(uam, a2f, z160)
        ufnf = _run_gru_fg(agg_fg, aggm, ufnf, p)

    at_s, at_m = _run_readout(uaf, agid_col, NAP)
    fg_s, fg_m = _run_readout(ufnf, fgid_col, NFP)
    return _run_head(at_s, at_m, fg_s, fg_m, mf, p)
